# Initial kernel scaffold; baseline (speedup 1.0000x reference)
import jax
import jax.numpy as jnp
from jax.experimental import pallas as pl


def kernel(x, edge_index, W1a, b1a, W2a, b2a, ga, ba, Ws1, bs1, Ws2, bs2, gs, bs):
    raise NotImplementedError("write your pallas kernel here")


# speedup vs baseline: 14.7311x; 14.7311x over previous
import functools

import jax
import jax.numpy as jnp
from jax import lax
from jax.experimental import pallas as pl
from jax.experimental.pallas import tpu as pltpu
from jax.experimental.pallas import tpu_sc as plsc

N = 10000
E = 320000
D_IN = 128
H = 16
L_EXTRA = 4

NC = 2
NS = 16
NW = NC * NS
PER_W = E // NW
N_PAD = 10112
ROWS_PER_TILE = N_PAD // NS


def _make_sc_agg(F, chunk):
    nchunk = PER_W // chunk
    zrows = ROWS_PER_TILE
    groups = F // H

    def body(h_hbm, src_hbm, dst_hbm, out_hbm, srcv, dstv, rows, acc, sem):
        cid = lax.axis_index("c")
        sid = lax.axis_index("s")

        def _z(i, _):
            for g in range(groups):
                rows[i, pl.ds(g * H, H)] = jnp.zeros((H,), jnp.float32)
            return 0

        lax.fori_loop(0, zrows, _z, 0)
        pltpu.sync_copy(rows.at[pl.ds(0, zrows)],
                        acc.at[pl.ds(sid * zrows, zrows)])
        plsc.subcore_barrier()

        base = (cid * NS + sid) * PER_W

        def _chunk(j, _):
            off = base + j * chunk
            pltpu.sync_copy(src_hbm.at[pl.ds(off, chunk)], srcv)
            pltpu.sync_copy(dst_hbm.at[pl.ds(off, chunk)], dstv)
            pltpu.async_copy(h_hbm.at[srcv], rows, sem).wait()
            pltpu.sync_copy(rows, acc.at[dstv], add=True)
            return 0

        lax.fori_loop(0, nchunk, _chunk, 0)
        plsc.subcore_barrier()

        pltpu.sync_copy(
            acc.at[pl.ds(sid * zrows, zrows)],
            out_hbm.at[cid, pl.ds(sid * zrows, zrows)],
        )

    return functools.partial(
        pl.kernel,
        out_type=jax.ShapeDtypeStruct((NC, N_PAD, F), jnp.float32),
        mesh=plsc.VectorSubcoreMesh(core_axis_name="c", subcore_axis_name="s"),
        scratch_types=[
            pltpu.VMEM((chunk,), jnp.int32),
            pltpu.VMEM((chunk,), jnp.int32),
            pltpu.VMEM((chunk, F), jnp.float32),
            pltpu.VMEM_SHARED((N_PAD, F), jnp.float32),
            pltpu.SemaphoreType.DMA,
        ],
        compiler_params=pltpu.CompilerParams(use_tc_tiling_on_sc=False),
    )(body)


_sc_agg_half = _make_sc_agg(D_IN // 2, 1000)
_sc_agg_narrow = _make_sc_agg(H, 2000)


def _mlp_bn(hin, w1, b1, w2, b2, g, beta):
    a = jnp.dot(hin, w1, preferred_element_type=jnp.float32) + b1
    a = jnp.maximum(a, 0.0)
    h = jnp.dot(a, w2, preferred_element_type=jnp.float32) + b2
    h = jnp.maximum(h, 0.0)
    mean = jnp.sum(h, axis=0, keepdims=True) * (1.0 / N)
    c = h - mean
    var = jnp.sum(c * c, axis=0, keepdims=True) * (1.0 / N)
    return c * lax.rsqrt(var + 1e-5) * g + beta


def _conv_body(h_ref, p_ref, w1_ref, b1_ref, w2_ref, b2_ref, g_ref, be_ref, o_ref):
    hin = h_ref[...] + p_ref[0, :N] + p_ref[1, :N]
    o_ref[...] = _mlp_bn(hin, w1_ref[...], b1_ref[...], w2_ref[...],
                         b2_ref[...], g_ref[...], be_ref[...])


def _conv1_body(h_ref, p0_ref, p1_ref, w1_ref, b1_ref, w2_ref, b2_ref, g_ref,
                be_ref, o_ref):
    s = jnp.concatenate(
        [p0_ref[0, :N] + p0_ref[1, :N], p1_ref[0, :N] + p1_ref[1, :N]], axis=1)
    hin = h_ref[...] + s
    o_ref[...] = _mlp_bn(hin, w1_ref[...], b1_ref[...], w2_ref[...],
                         b2_ref[...], g_ref[...], be_ref[...])


def _conv_last_body(h_ref, p_ref, w1_ref, b1_ref, w2_ref, b2_ref, g_ref, be_ref,
                    o_ref):
    hin = h_ref[...] + p_ref[0, :N] + p_ref[1, :N]
    z = _mlp_bn(hin, w1_ref[...], b1_ref[...], w2_ref[...], b2_ref[...],
                g_ref[...], be_ref[...])
    m = jnp.max(z, axis=1, keepdims=True)
    zs = z - m
    lse = jnp.log(jnp.sum(jnp.exp(zs), axis=1, keepdims=True))
    o_ref[...] = zs - lse


def _conv(h, parts, w1, b1, w2, b2, g, beta, last=False):
    return pl.pallas_call(
        _conv_last_body if last else _conv_body,
        out_shape=jax.ShapeDtypeStruct((N, H), jnp.float32),
    )(h, parts, w1, b1, w2, b2, g, beta)


def kernel(x, edge_index, W1a, b1a, W2a, b2a, ga, ba, Ws1, bs1, Ws2, bs2, gs, bs):
    src = edge_index[0].astype(jnp.int32)
    dst = edge_index[1].astype(jnp.int32)

    r = lambda v: v.reshape(1, H)

    x0 = x[:, : D_IN // 2] + 0.0
    x1 = x[:, D_IN // 2 :] + 0.0
    p0 = _sc_agg_half(x0, src, dst)
    p1 = _sc_agg_half(x1, src, dst)
    h = pl.pallas_call(
        _conv1_body,
        out_shape=jax.ShapeDtypeStruct((N, H), jnp.float32),
    )(x, p0, p1, W1a, r(b1a), W2a, r(b2a), r(ga), r(ba))
    for i in range(L_EXTRA):
        parts = _sc_agg_narrow(h, src, dst)
        h = _conv(h, parts, Ws1[i], r(bs1[i]), Ws2[i], r(bs2[i]), r(gs[i]),
                  r(bs[i]), last=(i == L_EXTRA - 1))
    return h
